# Initial kernel scaffold; baseline (speedup 1.0000x reference)
import jax
import jax.numpy as jnp
from jax.experimental import pallas as pl


def kernel(position, weight):
    raise NotImplementedError("write your pallas kernel here")


# speedup vs baseline: 1.9709x; 1.9709x over previous
import functools

import jax
import jax.numpy as jnp
from jax import lax
from jax.experimental import pallas as pl
from jax.experimental.pallas import tpu as pltpu
from jax.experimental.pallas import tpu_sc as plsc

NC, NS = 2, 16
NW = NC * NS
B = 32768
D = 1024
BPW = B // NW
C = 32
NCHUNK = BPW // C

_MESH = plsc.VectorSubcoreMesh(
    core_axis_name="c", subcore_axis_name="s", num_cores=NC, num_subcores=NS
)


@functools.partial(
    pl.kernel,
    out_type=jax.ShapeDtypeStruct((B, D), jnp.float32),
    mesh=_MESH,
    scratch_types=[
        pltpu.VMEM((NCHUNK, C), jnp.int32),
        pltpu.VMEM((C, D), jnp.float32),
        pltpu.SemaphoreType.DMA,
    ],
)
def _gather(pos_hbm, table_hbm, out_hbm, idx_v, rows_v, sem):
    wid = lax.axis_index("s") * NC + lax.axis_index("c")
    base = wid * BPW
    pltpu.sync_copy(pos_hbm.at[wid], idx_v)

    @pl.loop(0, NCHUNK)
    def _chunk(g):
        pltpu.async_copy(table_hbm.at[idx_v.at[g]], rows_v, sem).wait()
        pltpu.sync_copy(rows_v, out_hbm.at[pl.ds(base + g * C, C)])


def kernel(position, weight):
    pos3 = position.reshape(NW, NCHUNK, C).astype(jnp.int32)
    return _gather(pos3, weight)


# speedup vs baseline: 2.2815x; 1.1576x over previous
import functools

import jax
import jax.numpy as jnp
from jax import lax
from jax.experimental import pallas as pl
from jax.experimental.pallas import tpu as pltpu
from jax.experimental.pallas import tpu_sc as plsc

NC, NS = 2, 16
NW = NC * NS
B = 32768
D = 1024
BPW = B // NW
C = 32
NCHUNK = BPW // C

_MESH = plsc.VectorSubcoreMesh(
    core_axis_name="c", subcore_axis_name="s", num_cores=NC, num_subcores=NS
)


@functools.partial(
    pl.kernel,
    out_type=jax.ShapeDtypeStruct((B, D), jnp.float32),
    mesh=_MESH,
    scratch_types=[
        pltpu.VMEM((NCHUNK, C), jnp.int32),
        pltpu.VMEM((C, D), jnp.float32),
        pltpu.VMEM((C, D), jnp.float32),
        pltpu.SemaphoreType.DMA,
    ],
)
def _gather(pos_hbm, table_hbm, out_hbm, idx_v, rows0, rows1, gsem):
    wid = lax.axis_index("s") * NC + lax.axis_index("c")
    base = wid * BPW
    pltpu.sync_copy(pos_hbm.at[wid], idx_v)
    rows = (rows0, rows1)

    def start_gather(g, b):
        pltpu.async_copy(table_hbm.at[idx_v.at[g]], rows[b], gsem)

    def wait_gather(b):
        pltpu.make_async_copy(table_hbm.at[pl.ds(0, C)], rows[b], gsem).wait()

    def writeback(g, b):
        pltpu.sync_copy(rows[b], out_hbm.at[pl.ds(base + g * C, C)])

    start_gather(0, 0)

    @pl.loop(0, NCHUNK - 2, step=2)
    def _slot(g):
        for b in range(2):
            gg = g + b
            wait_gather(b)
            start_gather(gg + 1, 1 - b)
            writeback(gg, b)

    wait_gather(0)
    start_gather(NCHUNK - 1, 1)
    writeback(NCHUNK - 2, 0)
    wait_gather(1)
    writeback(NCHUNK - 1, 1)


def kernel(position, weight):
    pos3 = position.reshape(NW, NCHUNK, C).astype(jnp.int32)
    return _gather(pos3, weight)


# speedup vs baseline: 2.3414x; 1.0263x over previous
import functools

import jax
import jax.numpy as jnp
from jax import lax
from jax.experimental import pallas as pl
from jax.experimental.pallas import tpu as pltpu
from jax.experimental.pallas import tpu_sc as plsc

NC, NS = 2, 16
NW = NC * NS
B = 32768
D = 1024
BPW = B // NW
C = 32
NCHUNK = BPW // C

_MESH = plsc.VectorSubcoreMesh(
    core_axis_name="c", subcore_axis_name="s", num_cores=NC, num_subcores=NS
)


NBUF = 3


@functools.partial(
    pl.kernel,
    out_type=jax.ShapeDtypeStruct((B, D), jnp.float32),
    mesh=_MESH,
    scratch_types=[
        pltpu.VMEM((NCHUNK, C), jnp.int32),
        pltpu.VMEM((C, D), jnp.float32),
        pltpu.VMEM((C, D), jnp.float32),
        pltpu.VMEM((C, D), jnp.float32),
        pltpu.SemaphoreType.DMA,
        pltpu.SemaphoreType.DMA,
        pltpu.SemaphoreType.DMA,
        pltpu.SemaphoreType.DMA,
        pltpu.SemaphoreType.DMA,
        pltpu.SemaphoreType.DMA,
    ],
)
def _gather(pos_hbm, table_hbm, out_hbm, idx_v,
            rows0, rows1, rows2, g0, g1, g2, o0, o1, o2):
    wid = lax.axis_index("s") * NC + lax.axis_index("c")
    base = wid * BPW
    pltpu.sync_copy(pos_hbm.at[wid], idx_v)
    rows = (rows0, rows1, rows2)
    gsem = (g0, g1, g2)
    osem = (o0, o1, o2)

    def start_gather(g, b):
        pltpu.async_copy(table_hbm.at[idx_v.at[g]], rows[b], gsem[b])

    def wait_gather(b):
        pltpu.make_async_copy(table_hbm.at[pl.ds(0, C)], rows[b], gsem[b]).wait()

    def start_wb(g, b):
        pltpu.async_copy(rows[b], out_hbm.at[pl.ds(base + g * C, C)], osem[b])

    def wait_wb(b):
        pltpu.make_async_copy(rows[b], out_hbm.at[pl.ds(base, C)], osem[b]).wait()

    start_gather(0, 0)
    start_gather(1, 1)

    wait_gather(0)
    start_wb(0, 0)
    start_gather(2, 2)

    @pl.loop(1, NCHUNK - 4, step=NBUF)
    def _slot(g):
        for ss in range(NBUF):
            gg = g + ss
            b = (1 + ss) % NBUF
            wait_gather(b)
            start_wb(gg, b)
            bprev = (b - 1) % NBUF
            wait_wb(bprev)
            start_gather(gg + 2, bprev)

    for gg in (NCHUNK - 4, NCHUNK - 3):
        b = gg % NBUF
        wait_gather(b)
        start_wb(gg, b)
        bprev = (b - 1) % NBUF
        wait_wb(bprev)
        start_gather(gg + 2, bprev)

    for gg in (NCHUNK - 2, NCHUNK - 1):
        b = gg % NBUF
        wait_gather(b)
        start_wb(gg, b)

    for b in ((NCHUNK - 3) % NBUF, (NCHUNK - 2) % NBUF, (NCHUNK - 1) % NBUF):
        wait_wb(b)


def kernel(position, weight):
    pos3 = position.reshape(NW, NCHUNK, C).astype(jnp.int32)
    return _gather(pos3, weight)


# speedup vs baseline: 2.3529x; 1.0049x over previous
import functools

import jax
import jax.numpy as jnp
from jax import lax
from jax.experimental import pallas as pl
from jax.experimental.pallas import tpu as pltpu
from jax.experimental.pallas import tpu_sc as plsc

NC, NS = 2, 16
NW = NC * NS
B = 32768
D = 1024
BPW = B // NW
C = 16
NCHUNK = BPW // C
NBUF = 4

_MESH = plsc.VectorSubcoreMesh(
    core_axis_name="c", subcore_axis_name="s", num_cores=NC, num_subcores=NS
)


@functools.partial(
    pl.kernel,
    out_type=jax.ShapeDtypeStruct((B, D), jnp.float32),
    mesh=_MESH,
    scratch_types=[
        pltpu.VMEM((NCHUNK, C), jnp.int32),
        [pltpu.VMEM((C, D), jnp.float32) for _ in range(NBUF)],
        [pltpu.SemaphoreType.DMA for _ in range(NBUF)],
        [pltpu.SemaphoreType.DMA for _ in range(NBUF)],
    ],
)
def _gather(pos_hbm, table_hbm, out_hbm, idx_v, rows, gsem, osem):
    wid = lax.axis_index("s") * NC + lax.axis_index("c")
    base = wid * BPW
    pltpu.sync_copy(pos_hbm.at[wid], idx_v)

    def start_gather(g, b):
        pltpu.async_copy(table_hbm.at[idx_v.at[g]], rows[b], gsem[b])

    def wait_gather(b):
        pltpu.make_async_copy(table_hbm.at[pl.ds(0, C)], rows[b], gsem[b]).wait()

    def start_wb(g, b):
        pltpu.async_copy(rows[b], out_hbm.at[pl.ds(base + g * C, C)], osem[b])

    def wait_wb(b):
        pltpu.make_async_copy(rows[b], out_hbm.at[pl.ds(base, C)], osem[b]).wait()

    def slot(gg, b, lookahead, first_round):
        wait_gather(b)
        start_wb(gg, b)
        if lookahead:
            tbuf = (b - 1) % NBUF
            if not first_round:
                wait_wb(tbuf)
            start_gather(gg + NBUF - 1, tbuf)

    for k in range(NBUF - 1):
        start_gather(k, k)

    for gg in range(NBUF):
        slot(gg, gg % NBUF, True, gg == 0)

    last_la = NCHUNK - NBUF
    n_loop = (last_la + 1 - NBUF) // NBUF
    loop_end = NBUF + n_loop * NBUF

    @pl.loop(NBUF, loop_end, step=NBUF)
    def _rot(g):
        for ss in range(NBUF):
            slot(g + ss, ss, True, False)

    for gg in range(loop_end, last_la + 1):
        slot(gg, gg % NBUF, True, False)

    for gg in range(last_la + 1, NCHUNK):
        slot(gg, gg % NBUF, False, False)

    for g in range(NCHUNK - NBUF, NCHUNK):
        wait_wb(g % NBUF)


def kernel(position, weight):
    pos3 = position.reshape(NW, NCHUNK, C).astype(jnp.int32)
    return _gather(pos3, weight)


# speedup vs baseline: 2.3696x; 1.0071x over previous
import functools

import jax
import jax.numpy as jnp
from jax import lax
from jax.experimental import pallas as pl
from jax.experimental.pallas import tpu as pltpu
from jax.experimental.pallas import tpu_sc as plsc

NC, NS = 2, 16
NW = NC * NS
B = 32768
D = 1024
BPW = B // NW
C = 16
NCHUNK = BPW // C
NBUF = 4

_MESH = plsc.VectorSubcoreMesh(
    core_axis_name="c", subcore_axis_name="s", num_cores=NC, num_subcores=NS
)


@functools.partial(
    pl.kernel,
    out_type=jax.ShapeDtypeStruct((B, D), jnp.float32),
    mesh=_MESH,
    scratch_types=[
        pltpu.VMEM((BPW,), jnp.int32),
        [pltpu.VMEM((C, D), jnp.float32) for _ in range(NBUF)],
        [pltpu.SemaphoreType.DMA for _ in range(NBUF)],
        [pltpu.SemaphoreType.DMA for _ in range(NBUF)],
    ],
)
def _gather(pos_hbm, table_hbm, out_hbm, idx_v, rows, gsem, osem):
    wid = lax.axis_index("s") * NC + lax.axis_index("c")
    base = wid * BPW
    pltpu.sync_copy(pos_hbm.at[pl.ds(base, BPW)], idx_v)

    def start_gather(g, b):
        pltpu.async_copy(table_hbm.at[idx_v.at[pl.ds(g * C, C)]], rows[b], gsem[b])

    def wait_gather(b):
        pltpu.make_async_copy(table_hbm.at[pl.ds(0, C)], rows[b], gsem[b]).wait()

    def start_wb(g, b):
        pltpu.async_copy(rows[b], out_hbm.at[pl.ds(base + g * C, C)], osem[b])

    def wait_wb(b):
        pltpu.make_async_copy(rows[b], out_hbm.at[pl.ds(base, C)], osem[b]).wait()

    def slot(gg, b, lookahead, first_round):
        wait_gather(b)
        start_wb(gg, b)
        if lookahead:
            tbuf = (b - 1) % NBUF
            if not first_round:
                wait_wb(tbuf)
            start_gather(gg + NBUF - 1, tbuf)

    for k in range(NBUF - 1):
        start_gather(k, k)

    for gg in range(NBUF):
        slot(gg, gg % NBUF, True, gg == 0)

    last_la = NCHUNK - NBUF
    n_loop = (last_la + 1 - NBUF) // NBUF
    loop_end = NBUF + n_loop * NBUF

    @pl.loop(NBUF, loop_end, step=NBUF)
    def _rot(g):
        for ss in range(NBUF):
            slot(g + ss, ss, True, False)

    for gg in range(loop_end, last_la + 1):
        slot(gg, gg % NBUF, True, False)

    for gg in range(last_la + 1, NCHUNK):
        slot(gg, gg % NBUF, False, False)

    for g in range(NCHUNK - NBUF, NCHUNK):
        wait_wb(g % NBUF)


def kernel(position, weight):
    return _gather(position.astype(jnp.int32), weight)
